# baseline (speedup 1.0000x reference)
import functools

import jax
import jax.numpy as jnp
from jax import lax
from jax.experimental import pallas as pl
from jax.experimental.pallas import tpu as pltpu
from jax.experimental.pallas import tpu_sc as plsc

NC = 2
NS = 16
NW = NC * NS
CH = 128
WP = 8
NBUF = 4


def _sc_mesh():
    return plsc.VectorSubcoreMesh(core_axis_name="c", subcore_axis_name="s",
                                  num_cores=NC, num_subcores=NS)


def _make_deg_pass(NP, CPT):
    SL = NP // NS

    @functools.partial(
        pl.kernel,
        out_type=jax.ShapeDtypeStruct((NC * NP,), jnp.float32),
        mesh=_sc_mesh(),
        compiler_params=pltpu.CompilerParams(use_tc_tiling_on_sc=False),
        scratch_types=[
            pltpu.VMEM((CPT, CH), jnp.int32),
            pltpu.VMEM((CH,), jnp.float32),
            pltpu.VMEM((SL,), jnp.float32),
            pltpu.VMEM_SHARED((NP,), jnp.float32),
        ],
    )
    def deg_pass(dst_hbm, zero_hbm, out_hbm, dst_v, ones_v, bounce_v, deg_sh):
        cid = lax.axis_index("c")
        sid = lax.axis_index("s")
        wid = cid * NS + sid
        pltpu.sync_copy(dst_hbm.at[pl.ds(wid * CPT, CPT)], dst_v)
        for i in range(CH // 16):
            ones_v[pl.ds(i * 16, 16)] = jnp.full((16,), 1.0, jnp.float32)
        pltpu.sync_copy(zero_hbm.at[pl.ds(sid * SL, SL)], bounce_v)
        pltpu.sync_copy(bounce_v, deg_sh.at[pl.ds(sid * SL, SL)])
        plsc.subcore_barrier()

        def body(j, carry):
            pltpu.sync_copy(ones_v, deg_sh.at[dst_v.at[j]], add=True)
            return carry

        lax.fori_loop(0, CPT, body, 0)
        plsc.subcore_barrier()
        pltpu.sync_copy(deg_sh.at[pl.ds(sid * SL, SL)], bounce_v)
        pltpu.sync_copy(bounce_v, out_hbm.at[pl.ds(cid * NP + sid * SL, SL)])

    return deg_pass


def _make_edge_pass(NP, CPT):
    SL = NP // NS

    @functools.partial(
        pl.kernel,
        out_type=jax.ShapeDtypeStruct((NC * NP, WP), jnp.float32),
        mesh=_sc_mesh(),
        compiler_params=pltpu.CompilerParams(use_tc_tiling_on_sc=False),
        scratch_types=[
            pltpu.VMEM((CPT, CH), jnp.int32),
            pltpu.VMEM((CPT, CH), jnp.int32),
            pltpu.VMEM((NBUF, CH, WP), jnp.float32),
            pltpu.VMEM((SL, WP), jnp.float32),
            pltpu.SemaphoreType.DMA((NBUF,)),
            pltpu.VMEM_SHARED((NP, WP), jnp.float32),
            pltpu.VMEM_SHARED((NP, WP), jnp.float32),
        ],
    )
    def edge_pass(y_hbm, src_hbm, dst_hbm, zero_hbm, out_hbm,
                  src_v, dst_v, rows_v, bounce_v, gsem, acc_sh, y_sh):
        cid = lax.axis_index("c")
        sid = lax.axis_index("s")
        wid = cid * NS + sid
        pltpu.sync_copy(src_hbm.at[pl.ds(wid * CPT, CPT)], src_v)
        pltpu.sync_copy(dst_hbm.at[pl.ds(wid * CPT, CPT)], dst_v)
        pltpu.sync_copy(y_hbm.at[pl.ds(sid * SL, SL)], bounce_v)
        pltpu.sync_copy(bounce_v, y_sh.at[pl.ds(sid * SL, SL)])
        pltpu.sync_copy(zero_hbm.at[pl.ds(sid * SL, SL)], bounce_v)
        pltpu.sync_copy(bounce_v, acc_sh.at[pl.ds(sid * SL, SL)])
        plsc.subcore_barrier()

        for b in range(NBUF):
            pltpu.async_copy(y_sh.at[src_v.at[b]], rows_v.at[b], gsem.at[b])

        def body(i, carry):
            j0 = i * NBUF
            for b in range(NBUF):
                j = j0 + b
                pltpu.make_async_copy(y_sh.at[src_v.at[j]], rows_v.at[b],
                                      gsem.at[b]).wait()
                pltpu.sync_copy(rows_v.at[b], acc_sh.at[dst_v.at[j]], add=True)
                nj = j + NBUF

                @pl.when(nj < CPT)
                def _():
                    pltpu.async_copy(y_sh.at[src_v.at[nj]], rows_v.at[b],
                                     gsem.at[b])
            return carry

        lax.fori_loop(0, CPT // NBUF, body, 0)
        plsc.subcore_barrier()
        pltpu.sync_copy(acc_sh.at[pl.ds(sid * SL, SL)], bounce_v)
        pltpu.sync_copy(bounce_v, out_hbm.at[pl.ds(cid * NP + sid * SL, SL)])

    return edge_pass


def _tc_prep(deg0, deg1, x_pad, NP, BR):
    grid = NP // BR

    def body(d0, d1, xr, dis_o, y1_o):
        deg = d0[...] + d1[...] + 1.0
        dis = lax.rsqrt(deg)
        dis_o[...] = dis
        y1_o[...] = xr[...] * dis

    return pl.pallas_call(
        body,
        grid=(grid,),
        in_specs=[
            pl.BlockSpec((BR, 1), lambda i: (i, 0)),
            pl.BlockSpec((BR, 1), lambda i: (i, 0)),
            pl.BlockSpec((BR, WP), lambda i: (i, 0)),
        ],
        out_specs=[
            pl.BlockSpec((BR, 1), lambda i: (i, 0)),
            pl.BlockSpec((BR, WP), lambda i: (i, 0)),
        ],
        out_shape=[
            jax.ShapeDtypeStruct((NP, 1), jnp.float32),
            jax.ShapeDtypeStruct((NP, WP), jnp.float32),
        ],
    )(deg0, deg1, x_pad)


def _tc_dense(a0, a1, y1, dis, W1p, b1, W2p, NP, BR):
    grid = NP // BR

    def body(a0r, a1r, y1r, disr, W1r, b1r, W2r, y2_o):
        agg = (a0r[...] + a1r[...] + y1r[...]) * disr[...]
        h = jnp.dot(agg, W1r[...], preferred_element_type=jnp.float32)
        h = jnp.maximum(h + b1r[...], 0.0)
        p = jnp.dot(h, W2r[...], preferred_element_type=jnp.float32)
        y2_o[...] = p * disr[...]

    return pl.pallas_call(
        body,
        grid=(grid,),
        in_specs=[
            pl.BlockSpec((BR, WP), lambda i: (i, 0)),
            pl.BlockSpec((BR, WP), lambda i: (i, 0)),
            pl.BlockSpec((BR, WP), lambda i: (i, 0)),
            pl.BlockSpec((BR, 1), lambda i: (i, 0)),
            pl.BlockSpec((WP, 128), lambda i: (0, 0)),
            pl.BlockSpec((1, 128), lambda i: (0, 0)),
            pl.BlockSpec((128, WP), lambda i: (0, 0)),
        ],
        out_specs=pl.BlockSpec((BR, WP), lambda i: (i, 0)),
        out_shape=jax.ShapeDtypeStruct((NP, WP), jnp.float32),
    )(a0, a1, y1, dis, W1p, b1, W2p)


def _tc_final(a0, a1, y2, dis, b2, NP, BR, D_OUT):
    grid = NP // BR

    def body(a0r, a1r, y2r, disr, b2r, out_o):
        a16 = (a0r[...] + a1r[...] + y2r[...]) * disr[...]
        a = a16[:, :D_OUT] + b2r[...]
        m = jnp.max(a, axis=1, keepdims=True)
        lse = m + jnp.log(jnp.sum(jnp.exp(a - m), axis=1, keepdims=True))
        out_o[...] = a - lse

    return pl.pallas_call(
        body,
        grid=(grid,),
        in_specs=[
            pl.BlockSpec((BR, WP), lambda i: (i, 0)),
            pl.BlockSpec((BR, WP), lambda i: (i, 0)),
            pl.BlockSpec((BR, WP), lambda i: (i, 0)),
            pl.BlockSpec((BR, 1), lambda i: (i, 0)),
            pl.BlockSpec((1, D_OUT), lambda i: (0, 0)),
        ],
        out_specs=pl.BlockSpec((BR, D_OUT), lambda i: (i, 0)),
        out_shape=jax.ShapeDtypeStruct((NP, D_OUT), jnp.float32),
    )(a0, a1, y2, dis, b2)


def kernel(x, edge_index, W1, b1, W2, b2):
    N = x.shape[0]
    E = edge_index.shape[1]
    D_IN = x.shape[1]
    D_H = W1.shape[1]
    D_OUT = W2.shape[1]
    BR = 7168

    NP = ((N + 1 + 511) // 512) * 512

    CPT = -(-E // (NW * CH))
    CPT = -(-CPT // 8) * 8
    E_pad = NW * CPT * CH
    pad = E_pad - E
    src = jnp.concatenate([edge_index[0], jnp.full((pad,), N, jnp.int32)])
    dst = jnp.concatenate([edge_index[1], jnp.full((pad,), N, jnp.int32)])
    src2d = src.reshape(E_pad // CH, CH)
    dst2d = dst.reshape(E_pad // CH, CH)

    x_pad = jnp.pad(x, ((0, NP - N), (0, WP - D_IN)))
    W1p = jnp.pad(W1, ((0, WP - D_IN), (0, 0)))
    W2p = jnp.pad(W2, ((0, 0), (0, WP - D_OUT)))
    z1 = jnp.zeros((NP,), jnp.float32)
    zw = jnp.zeros((NP, WP), jnp.float32)

    deg_part = _make_deg_pass(NP, CPT)(dst2d, z1)
    dis, y1 = _tc_prep(deg_part[:NP, None], deg_part[NP:, None], x_pad,
                       NP, BR)
    edge_pass = _make_edge_pass(NP, CPT)
    acc1 = edge_pass(y1, src2d, dst2d, zw)
    y2 = _tc_dense(acc1[:NP], acc1[NP:], y1, dis, W1p, b1[None, :], W2p,
                   NP, BR)
    acc2 = edge_pass(y2, src2d, dst2d, zw)
    out = _tc_final(acc2[:NP], acc2[NP:], y2, dis, b2[None, :], NP, BR, D_OUT)
    return out[:N]


# speedup vs baseline: 1.0004x; 1.0004x over previous
import functools

import jax
import jax.numpy as jnp
from jax import lax
from jax.experimental import pallas as pl
from jax.experimental.pallas import tpu as pltpu
from jax.experimental.pallas import tpu_sc as plsc

NC = 2
NS = 16
NW = NC * NS
CH = 128
WP = 8
NBUF = 4


def _sc_mesh():
    return plsc.VectorSubcoreMesh(core_axis_name="c", subcore_axis_name="s",
                                  num_cores=NC, num_subcores=NS)


def _make_deg_pass(NP, CPT):
    SL = NP // NS

    @functools.partial(
        pl.kernel,
        out_type=jax.ShapeDtypeStruct((NC * NP,), jnp.float32),
        mesh=_sc_mesh(),
        compiler_params=pltpu.CompilerParams(use_tc_tiling_on_sc=False),
        scratch_types=[
            pltpu.VMEM((CPT, CH), jnp.int32),
            pltpu.VMEM((CH,), jnp.float32),
            pltpu.VMEM((SL,), jnp.float32),
            pltpu.VMEM_SHARED((NP,), jnp.float32),
        ],
    )
    def deg_pass(dst_hbm, zero_hbm, out_hbm, dst_v, ones_v, bounce_v, deg_sh):
        cid = lax.axis_index("c")
        sid = lax.axis_index("s")
        wid = cid * NS + sid
        pltpu.sync_copy(dst_hbm.at[pl.ds(wid * CPT, CPT)], dst_v)
        for i in range(CH // 16):
            ones_v[pl.ds(i * 16, 16)] = jnp.full((16,), 1.0, jnp.float32)
        pltpu.sync_copy(zero_hbm.at[pl.ds(sid * SL, SL)], bounce_v)
        pltpu.sync_copy(bounce_v, deg_sh.at[pl.ds(sid * SL, SL)])
        plsc.subcore_barrier()

        def body(j, carry):
            pltpu.sync_copy(ones_v, deg_sh.at[dst_v.at[j]], add=True)
            return carry

        lax.fori_loop(0, CPT, body, 0)
        plsc.subcore_barrier()
        pltpu.sync_copy(deg_sh.at[pl.ds(sid * SL, SL)], bounce_v)
        pltpu.sync_copy(bounce_v, out_hbm.at[pl.ds(cid * NP + sid * SL, SL)])

    return deg_pass


def _make_edge_pass(NP, CPT):
    SL = NP // NS

    @functools.partial(
        pl.kernel,
        out_type=jax.ShapeDtypeStruct((NC * NP, WP), jnp.float32),
        mesh=_sc_mesh(),
        compiler_params=pltpu.CompilerParams(use_tc_tiling_on_sc=False),
        scratch_types=[
            pltpu.VMEM((CPT, CH), jnp.int32),
            pltpu.VMEM((CPT, CH), jnp.int32),
            pltpu.VMEM((NBUF, CH, WP), jnp.float32),
            pltpu.VMEM((SL, WP), jnp.float32),
            pltpu.SemaphoreType.DMA((NBUF,)),
            pltpu.VMEM_SHARED((NP, WP), jnp.float32),
            pltpu.VMEM_SHARED((NP, WP), jnp.float32),
        ],
    )
    def edge_pass(y_hbm, src_hbm, dst_hbm, zero_hbm, out_hbm,
                  src_v, dst_v, rows_v, bounce_v, gsem, acc_sh, y_sh):
        cid = lax.axis_index("c")
        sid = lax.axis_index("s")
        wid = cid * NS + sid
        pltpu.sync_copy(src_hbm.at[pl.ds(wid * CPT, CPT)], src_v)
        pltpu.sync_copy(dst_hbm.at[pl.ds(wid * CPT, CPT)], dst_v)
        pltpu.sync_copy(y_hbm.at[pl.ds(sid * SL, SL)], bounce_v)
        pltpu.sync_copy(bounce_v, y_sh.at[pl.ds(sid * SL, SL)])
        pltpu.sync_copy(zero_hbm.at[pl.ds(sid * SL, SL)], bounce_v)
        pltpu.sync_copy(bounce_v, acc_sh.at[pl.ds(sid * SL, SL)])
        plsc.subcore_barrier()

        for b in range(NBUF):
            pltpu.async_copy(y_sh.at[src_v.at[b]], rows_v.at[b], gsem.at[b])

        def body(i, carry):
            j0 = i * NBUF
            for b in range(NBUF):
                j = j0 + b
                pltpu.make_async_copy(y_sh.at[src_v.at[j]], rows_v.at[b],
                                      gsem.at[b]).wait()
                pltpu.sync_copy(rows_v.at[b], acc_sh.at[dst_v.at[j]], add=True)
                nj = j + NBUF

                @pl.when(nj < CPT)
                def _():
                    pltpu.async_copy(y_sh.at[src_v.at[nj]], rows_v.at[b],
                                     gsem.at[b])
            return carry

        lax.fori_loop(0, CPT // NBUF, body, 0)
        plsc.subcore_barrier()
        pltpu.sync_copy(acc_sh.at[pl.ds(sid * SL, SL)], bounce_v)
        pltpu.sync_copy(bounce_v, out_hbm.at[pl.ds(cid * NP + sid * SL, SL)])

    return edge_pass


def _tc_prep(deg0, deg1, x_pad, NP, BR):
    grid = NP // BR

    def body(d0, d1, xr, dis_o, y1_o):
        deg = d0[...] + d1[...] + 1.0
        dis = lax.rsqrt(deg)
        dis_o[...] = dis
        y1_o[...] = xr[...] * dis

    return pl.pallas_call(
        body,
        grid=(grid,),
        in_specs=[
            pl.BlockSpec((BR, 1), lambda i: (i, 0)),
            pl.BlockSpec((BR, 1), lambda i: (i, 0)),
            pl.BlockSpec((BR, WP), lambda i: (i, 0)),
        ],
        out_specs=[
            pl.BlockSpec((BR, 1), lambda i: (i, 0)),
            pl.BlockSpec((BR, WP), lambda i: (i, 0)),
        ],
        out_shape=[
            jax.ShapeDtypeStruct((NP, 1), jnp.float32),
            jax.ShapeDtypeStruct((NP, WP), jnp.float32),
        ],
    )(deg0, deg1, x_pad)


def _tc_dense(a0, a1, y1, dis, W1p, b1, W2p, NP, BR):
    grid = NP // BR

    def body(a0r, a1r, y1r, disr, W1r, b1r, W2r, y2_o):
        agg = (a0r[...] + a1r[...] + y1r[...]) * disr[...]
        h = jnp.dot(agg, W1r[...], preferred_element_type=jnp.float32)
        h = jnp.maximum(h + b1r[...], 0.0)
        p = jnp.dot(h, W2r[...], preferred_element_type=jnp.float32)
        y2_o[...] = p * disr[...]

    return pl.pallas_call(
        body,
        grid=(grid,),
        in_specs=[
            pl.BlockSpec((BR, WP), lambda i: (i, 0)),
            pl.BlockSpec((BR, WP), lambda i: (i, 0)),
            pl.BlockSpec((BR, WP), lambda i: (i, 0)),
            pl.BlockSpec((BR, 1), lambda i: (i, 0)),
            pl.BlockSpec((WP, 128), lambda i: (0, 0)),
            pl.BlockSpec((1, 128), lambda i: (0, 0)),
            pl.BlockSpec((128, WP), lambda i: (0, 0)),
        ],
        out_specs=pl.BlockSpec((BR, WP), lambda i: (i, 0)),
        out_shape=jax.ShapeDtypeStruct((NP, WP), jnp.float32),
    )(a0, a1, y1, dis, W1p, b1, W2p)


def _tc_final(a0, a1, y2, dis, b2, NP, BR, D_OUT):
    grid = NP // BR

    def body(a0r, a1r, y2r, disr, b2r, out_o):
        a16 = (a0r[...] + a1r[...] + y2r[...]) * disr[...]
        a = a16[:, :D_OUT] + b2r[...]
        m = jnp.max(a, axis=1, keepdims=True)
        lse = m + jnp.log(jnp.sum(jnp.exp(a - m), axis=1, keepdims=True))
        out_o[...] = a - lse

    return pl.pallas_call(
        body,
        grid=(grid,),
        in_specs=[
            pl.BlockSpec((BR, WP), lambda i: (i, 0)),
            pl.BlockSpec((BR, WP), lambda i: (i, 0)),
            pl.BlockSpec((BR, WP), lambda i: (i, 0)),
            pl.BlockSpec((BR, 1), lambda i: (i, 0)),
            pl.BlockSpec((1, D_OUT), lambda i: (0, 0)),
        ],
        out_specs=pl.BlockSpec((BR, D_OUT), lambda i: (i, 0)),
        out_shape=jax.ShapeDtypeStruct((NP, D_OUT), jnp.float32),
    )(a0, a1, y2, dis, b2)


def kernel(x, edge_index, W1, b1, W2, b2):
    N = x.shape[0]
    E = edge_index.shape[1]
    D_IN = x.shape[1]
    D_H = W1.shape[1]
    D_OUT = W2.shape[1]
    BR = 3584

    NP = ((N + 1 + 511) // 512) * 512

    CPT = -(-E // (NW * CH))
    CPT = -(-CPT // 8) * 8
    E_pad = NW * CPT * CH
    pad = E_pad - E
    src = jnp.concatenate([edge_index[0], jnp.full((pad,), N, jnp.int32)])
    dst = jnp.concatenate([edge_index[1], jnp.full((pad,), N, jnp.int32)])
    src2d = src.reshape(E_pad // CH, CH)
    dst2d = dst.reshape(E_pad // CH, CH)

    x_pad = jnp.pad(x, ((0, NP - N), (0, WP - D_IN)))
    W1p = jnp.pad(W1, ((0, WP - D_IN), (0, 0)))
    W2p = jnp.pad(W2, ((0, 0), (0, WP - D_OUT)))
    z1 = jnp.zeros((NP,), jnp.float32)
    zw = jnp.zeros((NP, WP), jnp.float32)

    deg_part = _make_deg_pass(NP, CPT)(dst2d, z1)
    dis, y1 = _tc_prep(deg_part[:NP, None], deg_part[NP:, None], x_pad,
                       NP, BR)
    edge_pass = _make_edge_pass(NP, CPT)
    acc1 = edge_pass(y1, src2d, dst2d, zw)
    y2 = _tc_dense(acc1[:NP], acc1[NP:], y1, dis, W1p, b1[None, :], W2p,
                   NP, BR)
    acc2 = edge_pass(y2, src2d, dst2d, zw)
    out = _tc_final(acc2[:NP], acc2[NP:], y2, dis, b2[None, :], NP, BR, D_OUT)
    return out[:N]
